# baseline (speedup 1.0000x reference)
import functools

import jax
import jax.numpy as jnp
from jax import lax
from jax.experimental import pallas as pl
from jax.experimental.pallas import tpu as pltpu
from jax.experimental.pallas import tpu_sc as plsc

N = 10000
E = 160000
DF = 256
H = 512
T = 10
G = 128
BN_EPS = 1e-5

RB = 1000
NRB = N // RB
NCH = 4
CH = H // NCH

NSUB = 16
EPW = E // NSUB
EB = 128
NEB = 80
EPAD = NEB * EB
NST = 2
SB = NEB // NST
ACC_ROWS = 10240
PAD_DST = 10100
ZROWS = ACC_ROWS // NSUB

def _dot(a, b):
    return lax.dot_general(a, b, (((1,), (0,)), ((), ())),
                           preferred_element_type=jnp.float32)


def _mm(a, b):
    ah = a.astype(jnp.bfloat16)
    al = (a - ah.astype(jnp.float32)).astype(jnp.bfloat16)
    bh = b.astype(jnp.bfloat16)
    bl = (b - bh.astype(jnp.float32)).astype(jnp.bfloat16)
    return _dot(ah, bh) + (_dot(ah, bl) + _dot(al, bh))


def _stats_update(step, z, s_ref):
    @pl.when(step == 0)
    def _():
        s_ref[...] = jnp.zeros_like(s_ref)

    s_ref[0:1, :] += jnp.sum(z, axis=0, keepdims=True)
    s_ref[1:2, :] += jnp.sum(z * z, axis=0, keepdims=True)


def _bn_coefs(s_ref, g_ref, be_ref):
    m = s_ref[0:1, :] * (1.0 / N)
    ex2 = s_ref[1:2, :] * (1.0 / N)
    v = ex2 - m * m
    inv = lax.rsqrt(v + BN_EPS)
    a = g_ref[...] * inv
    c = be_ref[...] - m * a
    return a, c


def _passA0_body(x_ref, w_ref, b_ref, z_ref, s_ref):
    z = _mm(x_ref[...], w_ref[...]) + b_ref[...]
    z_ref[...] = z
    _stats_update(pl.program_id(0), z, s_ref)


def _passA0(x, w1, b1):
    return pl.pallas_call(
        _passA0_body,
        grid=(NRB,),
        in_specs=[
            pl.BlockSpec((RB, DF), lambda r: (r, 0)),
            pl.BlockSpec((DF, H), lambda r: (0, 0)),
            pl.BlockSpec((1, H), lambda r: (0, 0)),
        ],
        out_specs=[
            pl.BlockSpec((RB, H), lambda r: (r, 0)),
            pl.BlockSpec((8, H), lambda r: (0, 0)),
        ],
        out_shape=[
            jax.ShapeDtypeStruct((N, H), jnp.float32),
            jax.ShapeDtypeStruct((8, H), jnp.float32),
        ],
    )(x, w1, b1)


def _passA_body(h_ref, agg_ref, w_ref, b_ref, z_ref, s_ref):
    u = h_ref[...] + jnp.concatenate(
        [agg_ref[0], agg_ref[1], agg_ref[2], agg_ref[3]], axis=1)
    z = _mm(u, w_ref[...]) + b_ref[...]
    z_ref[...] = z
    _stats_update(pl.program_id(0), z, s_ref)


def _passA(h, agg4, w1, b1):
    return pl.pallas_call(
        _passA_body,
        grid=(NRB,),
        in_specs=[
            pl.BlockSpec((RB, H), lambda r: (r, 0)),
            pl.BlockSpec((NCH, RB, CH), lambda r: (0, r, 0)),
            pl.BlockSpec((H, H), lambda r: (0, 0)),
            pl.BlockSpec((1, H), lambda r: (0, 0)),
        ],
        out_specs=[
            pl.BlockSpec((RB, H), lambda r: (r, 0)),
            pl.BlockSpec((8, H), lambda r: (0, 0)),
        ],
        out_shape=[
            jax.ShapeDtypeStruct((N, H), jnp.float32),
            jax.ShapeDtypeStruct((8, H), jnp.float32),
        ],
    )(h, agg4, w1, b1)


def _passB_body(z1_ref, s1_ref, g_ref, be_ref, w_ref, b_ref, z2_ref, s2_ref):
    a, c = _bn_coefs(s1_ref, g_ref, be_ref)
    y = jnp.maximum(z1_ref[...] * a + c, 0.0)
    z2 = _mm(y, w_ref[...]) + b_ref[...]
    z2_ref[...] = z2
    _stats_update(pl.program_id(0), z2, s2_ref)


def _passB(z1, s1, g1, be1, w2, b2):
    return pl.pallas_call(
        _passB_body,
        grid=(NRB,),
        in_specs=[
            pl.BlockSpec((RB, H), lambda r: (r, 0)),
            pl.BlockSpec((8, H), lambda r: (0, 0)),
            pl.BlockSpec((1, H), lambda r: (0, 0)),
            pl.BlockSpec((1, H), lambda r: (0, 0)),
            pl.BlockSpec((H, H), lambda r: (0, 0)),
            pl.BlockSpec((1, H), lambda r: (0, 0)),
        ],
        out_specs=[
            pl.BlockSpec((RB, H), lambda r: (r, 0)),
            pl.BlockSpec((8, H), lambda r: (0, 0)),
        ],
        out_shape=[
            jax.ShapeDtypeStruct((N, H), jnp.float32),
            jax.ShapeDtypeStruct((8, H), jnp.float32),
        ],
    )(z1, s1, g1, be1, w2, b2)


def _passC_body(z2_ref, s2_ref, g_ref, be_ref, h_ref):
    a, c = _bn_coefs(s2_ref, g_ref, be_ref)
    h_ref[...] = jnp.maximum(z2_ref[...] * a + c, 0.0)


def _passC(z2, s2, g2, be2):
    return pl.pallas_call(
        _passC_body,
        grid=(NRB,),
        in_specs=[
            pl.BlockSpec((RB, H), lambda r: (r, 0)),
            pl.BlockSpec((8, H), lambda r: (0, 0)),
            pl.BlockSpec((1, H), lambda r: (0, 0)),
            pl.BlockSpec((1, H), lambda r: (0, 0)),
        ],
        out_specs=pl.BlockSpec((RB, H), lambda r: (r, 0)),
        out_shape=jax.ShapeDtypeStruct((N, H), jnp.float32),
    )(z2, s2, g2, be2)


def _passP_body(h_ref, b3_ref, p_ref):
    bb = b3_ref[0]
    maskT = (lax.broadcasted_iota(jnp.int32, (G, RB), 0) == bb
             ).astype(jnp.bfloat16)
    contrib = _dot(maskT, h_ref[...].astype(jnp.bfloat16))

    @pl.when(pl.program_id(0) == 0)
    def _():
        p_ref[...] = jnp.zeros_like(p_ref)

    p_ref[...] += contrib


def _passP(h, batch3):
    return pl.pallas_call(
        _passP_body,
        grid=(NRB,),
        in_specs=[
            pl.BlockSpec((RB, H), lambda r: (r, 0)),
            pl.BlockSpec((1, 1, RB), lambda r: (r, 0, 0)),
        ],
        out_specs=pl.BlockSpec((G, H), lambda r: (0, 0)),
        out_shape=jax.ShapeDtypeStruct((G, H), jnp.float32),
    )(h, batch3)


def _final_body(p5_ref, lw5_ref, lb5_ref, b3_ref, o_ref):
    acc = jnp.zeros((G, T), jnp.float32)
    for i in range(5):
        acc = acc + _mm(p5_ref[i], lw5_ref[i])
    cnt = jnp.zeros((G, 1), jnp.float32)
    for r in range(NRB):
        bb = b3_ref[r]
        mT = (lax.broadcasted_iota(jnp.int32, (G, RB), 0) == bb
              ).astype(jnp.float32)
        cnt = cnt + jnp.sum(mT, axis=1, keepdims=True)
    acc = acc + cnt * lb5_ref[0]
    acc = acc + (lb5_ref[1] + lb5_ref[2] + lb5_ref[3] + lb5_ref[4])
    o_ref[...] = acc


def _final(p5, lw5, lb5, batch3):
    return pl.pallas_call(
        _final_body,
        in_specs=[
            pl.BlockSpec((5, G, H), lambda: (0, 0, 0)),
            pl.BlockSpec((5, H, T), lambda: (0, 0, 0)),
            pl.BlockSpec((5, 1, T), lambda: (0, 0, 0)),
            pl.BlockSpec((NRB, 1, RB), lambda: (0, 0, 0)),
        ],
        out_specs=pl.BlockSpec((G, T), lambda: (0, 0)),
        out_shape=jax.ShapeDtypeStruct((G, T), jnp.float32),
    )(p5, lw5, lb5, batch3)


def _sc_agg(h2d, sidx4, didx):
    mesh = plsc.VectorSubcoreMesh(core_axis_name="c", subcore_axis_name="s")

    @functools.partial(
        pl.kernel, mesh=mesh,
        out_type=jax.ShapeDtypeStruct((NCH, ACC_ROWS, CH), jnp.float32),
        scratch_types=[
            pltpu.VMEM((SB, EB), jnp.int32),
            pltpu.VMEM((SB, EB), jnp.int32),
            pltpu.VMEM((EB, CH), jnp.float32),
            pltpu.VMEM((EB, CH), jnp.float32),
            pltpu.SemaphoreType.DMA,
            pltpu.SemaphoreType.DMA,
            pltpu.SemaphoreType.DMA,
            pltpu.SemaphoreType.DMA,
            pltpu.VMEM_SHARED((ACC_ROWS, CH), jnp.float32),
        ],
    )
    def k(h_hbm, sidx_hbm, didx_hbm, out_hbm,
          sbuf, dbuf, rbuf0, rbuf1, sem0, sem1, gsem0, gsem1, acc):
        cid = lax.axis_index("c")
        sid = lax.axis_index("s")
        rbufs = (rbuf0, rbuf1)
        sems = (sem0, sem1)
        gsems = (gsem0, gsem1)

        def wait_scatter(b):
            pltpu.make_async_copy(rbufs[b], acc.at[dbuf.at[0]],
                                  sems[b]).wait()

        def wait_gather(b):
            pltpu.make_async_copy(h_hbm.at[sbuf.at[0]], rbufs[b],
                                  gsems[b]).wait()

        for cc in range(2):
            chunk = cid * 2 + cc
            base = sid * ZROWS

            @pl.loop(0, EB)
            def _(i):
                for kk in range(CH // 16):
                    rbuf0[i, pl.ds(kk * 16, 16)] = jnp.zeros((16,),
                                                             jnp.float32)
            for kk in range(ZROWS // EB):
                pltpu.sync_copy(rbuf0, acc.at[pl.ds(base + kk * EB, EB)])
            plsc.subcore_barrier()

            for st in range(NST):
                pltpu.sync_copy(sidx_hbm.at[sid].at[pl.ds(st * SB, SB)],
                                sbuf)
                pltpu.sync_copy(didx_hbm.at[sid].at[pl.ds(st * SB, SB)],
                                dbuf)

                @pl.loop(0, SB)
                def _(j):
                    for kk in range(EB // 16):
                        sbuf[j, pl.ds(kk * 16, 16)] = (
                            sbuf[j, pl.ds(kk * 16, 16)] + chunk)

                pltpu.async_copy(h_hbm.at[sbuf.at[0]], rbuf0, gsem0)

                def step(j, u, prefetch, guard_first=False):
                    b = u % 2
                    nb = 1 - b
                    wait_gather(b)
                    if prefetch:
                        if guard_first:
                            @pl.when(j > 0)
                            def _():
                                wait_scatter(nb)
                        else:
                            wait_scatter(nb)
                        pltpu.async_copy(h_hbm.at[sbuf.at[j + 1]],
                                         rbufs[nb], gsems[nb])
                    pltpu.async_copy(rbufs[b], acc.at[dbuf.at[j]],
                                     sems[b], add=True)

                @pl.loop(0, SB // 4 - 1)
                def _(i):
                    for u in range(4):
                        step(i * 4 + u, u, True, guard_first=(u == 0))

                for u in range(4):
                    step(SB - 4 + u, u, u < 3)

                wait_scatter(0)
                wait_scatter(1)

            plsc.subcore_barrier()
            pltpu.sync_copy(
                acc.at[pl.ds(sid * ZROWS, ZROWS)],
                out_hbm.at[chunk].at[pl.ds(sid * ZROWS, ZROWS)])
            plsc.subcore_barrier()

    return k(h2d, sidx4, didx)


def _mlp0(x, p):
    z1, s1 = _passA0(x, p["w1"], p["b1"].reshape(1, H))
    z2, s2 = _passB(z1, s1, p["g1"].reshape(1, H), p["be1"].reshape(1, H),
                    p["w2"], p["b2"].reshape(1, H))
    return _passC(z2, s2, p["g2"].reshape(1, H), p["be2"].reshape(1, H))


def _mlp(h, agg4, p):
    z1, s1 = _passA(h, agg4, p["w1"], p["b1"].reshape(1, H))
    z2, s2 = _passB(z1, s1, p["g1"].reshape(1, H), p["be1"].reshape(1, H),
                    p["w2"], p["b2"].reshape(1, H))
    return _passC(z2, s2, p["g2"].reshape(1, H), p["be2"].reshape(1, H))


def kernel(x, edge_index, batch, params):
    src = edge_index[0].astype(jnp.int32)
    dst = edge_index[1].astype(jnp.int32)
    npad = EPAD - EPW
    ar = jnp.arange(npad, dtype=jnp.int32)[None, :]
    aw = jnp.arange(NSUB, dtype=jnp.int32)[:, None]
    pad_src = (ar * 163 + aw * 613) % N
    pad_dst = N + (ar + 15 * aw) % (ACC_ROWS - N)
    srcp = jnp.concatenate([src.reshape(NSUB, EPW), pad_src], axis=1)
    dstp = jnp.concatenate([dst.reshape(NSUB, EPW), pad_dst], axis=1)
    sidx4 = (srcp * NCH).reshape(NSUB, NEB, EB)
    didx = dstp.reshape(NSUB, NEB, EB)
    batch3 = batch.astype(jnp.int32).reshape(NRB, 1, RB)

    h = _mlp0(x, params["first_h"])
    pooled = []
    for i in range(4):
        agg4 = _sc_agg(h.reshape(N * NCH, CH), sidx4, didx)
        pooled.append(_passP(h, batch3))
        h = _mlp(h, agg4, params["nns"][i])
    pooled.append(_passP(h, batch3))

    p5 = jnp.stack(pooled)
    lw5 = jnp.stack(params["lin_w"])
    lb5 = jnp.stack(params["lin_b"]).reshape(5, 1, T)
    return _final(p5, lw5, lb5, batch3)


# speedup vs baseline: 1.0099x; 1.0099x over previous
import functools

import jax
import jax.numpy as jnp
from jax import lax
from jax.experimental import pallas as pl
from jax.experimental.pallas import tpu as pltpu
from jax.experimental.pallas import tpu_sc as plsc

N = 10000
E = 160000
DF = 256
H = 512
T = 10
G = 128
BN_EPS = 1e-5

RB = 1000
NRB = N // RB
NCH = 4
CH = H // NCH

NSUB = 16
EPW = E // NSUB
EB = 128
NEB = 80
EPAD = NEB * EB
NST = 2
SB = NEB // NST
ACC_ROWS = 10240
PAD_DST = 10100
ZROWS = ACC_ROWS // NSUB

def _dot(a, b):
    return lax.dot_general(a, b, (((1,), (0,)), ((), ())),
                           preferred_element_type=jnp.float32)


def _mm(a, b):
    ah = a.astype(jnp.bfloat16)
    al = (a - ah.astype(jnp.float32)).astype(jnp.bfloat16)
    bh = b.astype(jnp.bfloat16)
    bl = (b - bh.astype(jnp.float32)).astype(jnp.bfloat16)
    return _dot(ah, bh) + (_dot(ah, bl) + _dot(al, bh))


def _stats_update(step, z, s_ref):
    @pl.when(step == 0)
    def _():
        s_ref[...] = jnp.zeros_like(s_ref)

    s_ref[0:1, :] += jnp.sum(z, axis=0, keepdims=True)
    s_ref[1:2, :] += jnp.sum(z * z, axis=0, keepdims=True)


def _bn_coefs(s_ref, g_ref, be_ref):
    m = s_ref[0:1, :] * (1.0 / N)
    ex2 = s_ref[1:2, :] * (1.0 / N)
    v = ex2 - m * m
    inv = lax.rsqrt(v + BN_EPS)
    a = g_ref[...] * inv
    c = be_ref[...] - m * a
    return a, c


def _passA0_body(x_ref, w_ref, b_ref, z_ref, s_ref):
    z = _mm(x_ref[...], w_ref[...]) + b_ref[...]
    z_ref[...] = z
    _stats_update(pl.program_id(0), z, s_ref)


def _passA0(x, w1, b1):
    return pl.pallas_call(
        _passA0_body,
        grid=(NRB,),
        in_specs=[
            pl.BlockSpec((RB, DF), lambda r: (r, 0)),
            pl.BlockSpec((DF, H), lambda r: (0, 0)),
            pl.BlockSpec((1, H), lambda r: (0, 0)),
        ],
        out_specs=[
            pl.BlockSpec((RB, H), lambda r: (r, 0)),
            pl.BlockSpec((8, H), lambda r: (0, 0)),
        ],
        out_shape=[
            jax.ShapeDtypeStruct((N, H), jnp.float32),
            jax.ShapeDtypeStruct((8, H), jnp.float32),
        ],
    )(x, w1, b1)


def _passA_body(h_ref, agg_ref, w_ref, b_ref, z_ref, s_ref):
    u = h_ref[...] + jnp.concatenate(
        [agg_ref[0], agg_ref[1], agg_ref[2], agg_ref[3]], axis=1)
    z = _mm(u, w_ref[...]) + b_ref[...]
    z_ref[...] = z
    _stats_update(pl.program_id(0), z, s_ref)


def _passA(h, agg4, w1, b1):
    return pl.pallas_call(
        _passA_body,
        grid=(NRB,),
        in_specs=[
            pl.BlockSpec((RB, H), lambda r: (r, 0)),
            pl.BlockSpec((NCH, RB, CH), lambda r: (0, r, 0)),
            pl.BlockSpec((H, H), lambda r: (0, 0)),
            pl.BlockSpec((1, H), lambda r: (0, 0)),
        ],
        out_specs=[
            pl.BlockSpec((RB, H), lambda r: (r, 0)),
            pl.BlockSpec((8, H), lambda r: (0, 0)),
        ],
        out_shape=[
            jax.ShapeDtypeStruct((N, H), jnp.float32),
            jax.ShapeDtypeStruct((8, H), jnp.float32),
        ],
    )(h, agg4, w1, b1)


def _passB_body(z1_ref, s1_ref, g_ref, be_ref, w_ref, b_ref, z2_ref, s2_ref):
    a, c = _bn_coefs(s1_ref, g_ref, be_ref)
    y = jnp.maximum(z1_ref[...] * a + c, 0.0)
    z2 = _mm(y, w_ref[...]) + b_ref[...]
    z2_ref[...] = z2
    _stats_update(pl.program_id(0), z2, s2_ref)


def _passB(z1, s1, g1, be1, w2, b2):
    return pl.pallas_call(
        _passB_body,
        grid=(NRB,),
        in_specs=[
            pl.BlockSpec((RB, H), lambda r: (r, 0)),
            pl.BlockSpec((8, H), lambda r: (0, 0)),
            pl.BlockSpec((1, H), lambda r: (0, 0)),
            pl.BlockSpec((1, H), lambda r: (0, 0)),
            pl.BlockSpec((H, H), lambda r: (0, 0)),
            pl.BlockSpec((1, H), lambda r: (0, 0)),
        ],
        out_specs=[
            pl.BlockSpec((RB, H), lambda r: (r, 0)),
            pl.BlockSpec((8, H), lambda r: (0, 0)),
        ],
        out_shape=[
            jax.ShapeDtypeStruct((N, H), jnp.float32),
            jax.ShapeDtypeStruct((8, H), jnp.float32),
        ],
    )(z1, s1, g1, be1, w2, b2)


def _passC_body(z2_ref, s2_ref, g_ref, be_ref, b3_ref, h_ref, p_ref):
    a, c = _bn_coefs(s2_ref, g_ref, be_ref)
    y = jnp.maximum(z2_ref[...] * a + c, 0.0)
    h_ref[...] = y
    bb = b3_ref[0]
    maskT = (lax.broadcasted_iota(jnp.int32, (G, RB), 0) == bb
             ).astype(jnp.bfloat16)
    contrib = _dot(maskT, y.astype(jnp.bfloat16))

    @pl.when(pl.program_id(0) == 0)
    def _():
        p_ref[...] = jnp.zeros_like(p_ref)

    p_ref[...] += contrib


def _passC(z2, s2, g2, be2, batch3):
    return pl.pallas_call(
        _passC_body,
        grid=(NRB,),
        in_specs=[
            pl.BlockSpec((RB, H), lambda r: (r, 0)),
            pl.BlockSpec((8, H), lambda r: (0, 0)),
            pl.BlockSpec((1, H), lambda r: (0, 0)),
            pl.BlockSpec((1, H), lambda r: (0, 0)),
            pl.BlockSpec((1, 1, RB), lambda r: (r, 0, 0)),
        ],
        out_specs=[
            pl.BlockSpec((RB, H), lambda r: (r, 0)),
            pl.BlockSpec((G, H), lambda r: (0, 0)),
        ],
        out_shape=[
            jax.ShapeDtypeStruct((N, H), jnp.float32),
            jax.ShapeDtypeStruct((G, H), jnp.float32),
        ],
    )(z2, s2, g2, be2, batch3)


def _final_body(p5_ref, lw5_ref, lb5_ref, b3_ref, o_ref):
    acc = jnp.zeros((G, T), jnp.float32)
    for i in range(5):
        acc = acc + _mm(p5_ref[i], lw5_ref[i])
    cnt = jnp.zeros((G, 1), jnp.float32)
    for r in range(NRB):
        bb = b3_ref[r]
        mT = (lax.broadcasted_iota(jnp.int32, (G, RB), 0) == bb
              ).astype(jnp.float32)
        cnt = cnt + jnp.sum(mT, axis=1, keepdims=True)
    acc = acc + cnt * lb5_ref[0]
    acc = acc + (lb5_ref[1] + lb5_ref[2] + lb5_ref[3] + lb5_ref[4])
    o_ref[...] = acc


def _final(p5, lw5, lb5, batch3):
    return pl.pallas_call(
        _final_body,
        in_specs=[
            pl.BlockSpec((5, G, H), lambda: (0, 0, 0)),
            pl.BlockSpec((5, H, T), lambda: (0, 0, 0)),
            pl.BlockSpec((5, 1, T), lambda: (0, 0, 0)),
            pl.BlockSpec((NRB, 1, RB), lambda: (0, 0, 0)),
        ],
        out_specs=pl.BlockSpec((G, T), lambda: (0, 0)),
        out_shape=jax.ShapeDtypeStruct((G, T), jnp.float32),
    )(p5, lw5, lb5, batch3)


def _sc_agg(h2d, sidx4, didx):
    mesh = plsc.VectorSubcoreMesh(core_axis_name="c", subcore_axis_name="s")

    @functools.partial(
        pl.kernel, mesh=mesh,
        out_type=jax.ShapeDtypeStruct((NCH, ACC_ROWS, CH), jnp.float32),
        scratch_types=[
            pltpu.VMEM((SB, EB), jnp.int32),
            pltpu.VMEM((SB, EB), jnp.int32),
            pltpu.VMEM((EB, CH), jnp.float32),
            pltpu.VMEM((EB, CH), jnp.float32),
            pltpu.SemaphoreType.DMA,
            pltpu.SemaphoreType.DMA,
            pltpu.SemaphoreType.DMA,
            pltpu.SemaphoreType.DMA,
            pltpu.VMEM_SHARED((ACC_ROWS, CH), jnp.float32),
        ],
    )
    def k(h_hbm, sidx_hbm, didx_hbm, out_hbm,
          sbuf, dbuf, rbuf0, rbuf1, sem0, sem1, gsem0, gsem1, acc):
        cid = lax.axis_index("c")
        sid = lax.axis_index("s")
        rbufs = (rbuf0, rbuf1)
        sems = (sem0, sem1)
        gsems = (gsem0, gsem1)

        def wait_scatter(b):
            pltpu.make_async_copy(rbufs[b], acc.at[dbuf.at[0]],
                                  sems[b]).wait()

        def wait_gather(b):
            pltpu.make_async_copy(h_hbm.at[sbuf.at[0]], rbufs[b],
                                  gsems[b]).wait()

        for cc in range(2):
            chunk = cid * 2 + cc
            base = sid * ZROWS

            @pl.loop(0, EB)
            def _(i):
                for kk in range(CH // 16):
                    rbuf0[i, pl.ds(kk * 16, 16)] = jnp.zeros((16,),
                                                             jnp.float32)
            for kk in range(ZROWS // EB):
                pltpu.sync_copy(rbuf0, acc.at[pl.ds(base + kk * EB, EB)])
            plsc.subcore_barrier()

            for st in range(NST):
                pltpu.sync_copy(sidx_hbm.at[sid].at[pl.ds(st * SB, SB)],
                                sbuf)
                pltpu.sync_copy(didx_hbm.at[sid].at[pl.ds(st * SB, SB)],
                                dbuf)

                @pl.loop(0, SB)
                def _(j):
                    for kk in range(EB // 16):
                        sbuf[j, pl.ds(kk * 16, 16)] = (
                            sbuf[j, pl.ds(kk * 16, 16)] + chunk)

                pltpu.async_copy(h_hbm.at[sbuf.at[0]], rbuf0, gsem0)

                def step(j, u, prefetch, guard_first=False):
                    b = u % 2
                    nb = 1 - b
                    wait_gather(b)
                    if prefetch:
                        if guard_first:
                            @pl.when(j > 0)
                            def _():
                                wait_scatter(nb)
                        else:
                            wait_scatter(nb)
                        pltpu.async_copy(h_hbm.at[sbuf.at[j + 1]],
                                         rbufs[nb], gsems[nb])
                    pltpu.async_copy(rbufs[b], acc.at[dbuf.at[j]],
                                     sems[b], add=True)

                @pl.loop(0, SB // 4 - 1)
                def _(i):
                    for u in range(4):
                        step(i * 4 + u, u, True, guard_first=(u == 0))

                for u in range(4):
                    step(SB - 4 + u, u, u < 3)

                wait_scatter(0)
                wait_scatter(1)

            plsc.subcore_barrier()
            pltpu.sync_copy(
                acc.at[pl.ds(sid * ZROWS, ZROWS)],
                out_hbm.at[chunk].at[pl.ds(sid * ZROWS, ZROWS)])
            plsc.subcore_barrier()

    return k(h2d, sidx4, didx)


def _mlp0(x, p, batch3):
    z1, s1 = _passA0(x, p["w1"], p["b1"].reshape(1, H))
    z2, s2 = _passB(z1, s1, p["g1"].reshape(1, H), p["be1"].reshape(1, H),
                    p["w2"], p["b2"].reshape(1, H))
    return _passC(z2, s2, p["g2"].reshape(1, H), p["be2"].reshape(1, H),
                  batch3)


def _mlp(h, agg4, p, batch3):
    z1, s1 = _passA(h, agg4, p["w1"], p["b1"].reshape(1, H))
    z2, s2 = _passB(z1, s1, p["g1"].reshape(1, H), p["be1"].reshape(1, H),
                    p["w2"], p["b2"].reshape(1, H))
    return _passC(z2, s2, p["g2"].reshape(1, H), p["be2"].reshape(1, H),
                  batch3)


def kernel(x, edge_index, batch, params):
    src = edge_index[0].astype(jnp.int32)
    dst = edge_index[1].astype(jnp.int32)
    npad = EPAD - EPW
    ar = jnp.arange(npad, dtype=jnp.int32)[None, :]
    aw = jnp.arange(NSUB, dtype=jnp.int32)[:, None]
    pad_src = (ar * 163 + aw * 613) % N
    pad_dst = N + (ar + 15 * aw) % (ACC_ROWS - N)
    srcp = jnp.concatenate([src.reshape(NSUB, EPW), pad_src], axis=1)
    dstp = jnp.concatenate([dst.reshape(NSUB, EPW), pad_dst], axis=1)
    sidx4 = (srcp * NCH).reshape(NSUB, NEB, EB)
    didx = dstp.reshape(NSUB, NEB, EB)
    batch3 = batch.astype(jnp.int32).reshape(NRB, 1, RB)

    h, pooled0 = _mlp0(x, params["first_h"], batch3)
    pooled = [pooled0]
    for i in range(4):
        agg4 = _sc_agg(h.reshape(N * NCH, CH), sidx4, didx)
        h, p_i = _mlp(h, agg4, params["nns"][i], batch3)
        pooled.append(p_i)

    p5 = jnp.stack(pooled)
    lw5 = jnp.stack(params["lin_w"])
    lb5 = jnp.stack(params["lin_b"]).reshape(5, 1, T)
    return _final(p5, lw5, lb5, batch3)


# speedup vs baseline: 1.0211x; 1.0110x over previous
import functools

import jax
import jax.numpy as jnp
from jax import lax
from jax.experimental import pallas as pl
from jax.experimental.pallas import tpu as pltpu
from jax.experimental.pallas import tpu_sc as plsc

N = 10000
E = 160000
DF = 256
H = 512
T = 10
G = 128
BN_EPS = 1e-5

RB = 2000
NRB = N // RB
NCH = 4
CH = H // NCH

NSUB = 16
EPW = E // NSUB
EB = 128
NEB = 80
EPAD = NEB * EB
NST = 2
SB = NEB // NST
ACC_ROWS = 10240
ZROWS = ACC_ROWS // NSUB

def _dot(a, b):
    return lax.dot_general(a, b, (((1,), (0,)), ((), ())),
                           preferred_element_type=jnp.float32)


def _mm(a, b):
    ah = a.astype(jnp.bfloat16)
    al = (a - ah.astype(jnp.float32)).astype(jnp.bfloat16)
    bh = b.astype(jnp.bfloat16)
    bl = (b - bh.astype(jnp.float32)).astype(jnp.bfloat16)
    return _dot(ah, bh) + (_dot(ah, bl) + _dot(al, bh))


def _stats_update(step, z, s_ref):
    @pl.when(step == 0)
    def _():
        s_ref[...] = jnp.zeros_like(s_ref)

    s_ref[0:1, :] += jnp.sum(z, axis=0, keepdims=True)
    s_ref[1:2, :] += jnp.sum(z * z, axis=0, keepdims=True)


def _bn_coefs(s_ref, g_ref, be_ref):
    m = s_ref[0:1, :] * (1.0 / N)
    ex2 = s_ref[1:2, :] * (1.0 / N)
    v = ex2 - m * m
    inv = lax.rsqrt(v + BN_EPS)
    a = g_ref[...] * inv
    c = be_ref[...] - m * a
    return a, c


def _passA0_body(x_ref, w_ref, b_ref, z_ref, s_ref):
    z = _mm(x_ref[...], w_ref[...]) + b_ref[...]
    z_ref[...] = z
    _stats_update(pl.program_id(0), z, s_ref)


def _passA0(x, w1, b1):
    return pl.pallas_call(
        _passA0_body,
        grid=(NRB,),
        in_specs=[
            pl.BlockSpec((RB, DF), lambda r: (r, 0)),
            pl.BlockSpec((DF, H), lambda r: (0, 0)),
            pl.BlockSpec((1, H), lambda r: (0, 0)),
        ],
        out_specs=[
            pl.BlockSpec((RB, H), lambda r: (r, 0)),
            pl.BlockSpec((8, H), lambda r: (0, 0)),
        ],
        out_shape=[
            jax.ShapeDtypeStruct((N, H), jnp.float32),
            jax.ShapeDtypeStruct((8, H), jnp.float32),
        ],
    )(x, w1, b1)


def _passA_body(h_ref, agg_ref, w_ref, b_ref, z_ref, s_ref):
    u = h_ref[...] + jnp.concatenate(
        [agg_ref[0], agg_ref[1], agg_ref[2], agg_ref[3]], axis=1)
    z = _mm(u, w_ref[...]) + b_ref[...]
    z_ref[...] = z
    _stats_update(pl.program_id(0), z, s_ref)


def _passA(h, agg4, w1, b1):
    return pl.pallas_call(
        _passA_body,
        grid=(NRB,),
        in_specs=[
            pl.BlockSpec((RB, H), lambda r: (r, 0)),
            pl.BlockSpec((NCH, RB, CH), lambda r: (0, r, 0)),
            pl.BlockSpec((H, H), lambda r: (0, 0)),
            pl.BlockSpec((1, H), lambda r: (0, 0)),
        ],
        out_specs=[
            pl.BlockSpec((RB, H), lambda r: (r, 0)),
            pl.BlockSpec((8, H), lambda r: (0, 0)),
        ],
        out_shape=[
            jax.ShapeDtypeStruct((N, H), jnp.float32),
            jax.ShapeDtypeStruct((8, H), jnp.float32),
        ],
    )(h, agg4, w1, b1)


def _passB_body(z1_ref, s1_ref, g_ref, be_ref, w_ref, b_ref, z2_ref, s2_ref):
    a, c = _bn_coefs(s1_ref, g_ref, be_ref)
    y = jnp.maximum(z1_ref[...] * a + c, 0.0)
    z2 = _mm(y, w_ref[...]) + b_ref[...]
    z2_ref[...] = z2
    _stats_update(pl.program_id(0), z2, s2_ref)


def _passB(z1, s1, g1, be1, w2, b2):
    return pl.pallas_call(
        _passB_body,
        grid=(NRB,),
        in_specs=[
            pl.BlockSpec((RB, H), lambda r: (r, 0)),
            pl.BlockSpec((8, H), lambda r: (0, 0)),
            pl.BlockSpec((1, H), lambda r: (0, 0)),
            pl.BlockSpec((1, H), lambda r: (0, 0)),
            pl.BlockSpec((H, H), lambda r: (0, 0)),
            pl.BlockSpec((1, H), lambda r: (0, 0)),
        ],
        out_specs=[
            pl.BlockSpec((RB, H), lambda r: (r, 0)),
            pl.BlockSpec((8, H), lambda r: (0, 0)),
        ],
        out_shape=[
            jax.ShapeDtypeStruct((N, H), jnp.float32),
            jax.ShapeDtypeStruct((8, H), jnp.float32),
        ],
    )(z1, s1, g1, be1, w2, b2)


def _passC_body(z2_ref, s2_ref, g_ref, be_ref, b3_ref, h_ref, p_ref):
    a, c = _bn_coefs(s2_ref, g_ref, be_ref)
    y = jnp.maximum(z2_ref[...] * a + c, 0.0)
    h_ref[...] = y
    bb = b3_ref[0]
    maskT = (lax.broadcasted_iota(jnp.int32, (G, RB), 0) == bb
             ).astype(jnp.bfloat16)
    contrib = _dot(maskT, y.astype(jnp.bfloat16))

    @pl.when(pl.program_id(0) == 0)
    def _():
        p_ref[...] = jnp.zeros_like(p_ref)

    p_ref[...] += contrib


def _passC(z2, s2, g2, be2, batch3):
    return pl.pallas_call(
        _passC_body,
        grid=(NRB,),
        in_specs=[
            pl.BlockSpec((RB, H), lambda r: (r, 0)),
            pl.BlockSpec((8, H), lambda r: (0, 0)),
            pl.BlockSpec((1, H), lambda r: (0, 0)),
            pl.BlockSpec((1, H), lambda r: (0, 0)),
            pl.BlockSpec((1, 1, RB), lambda r: (r, 0, 0)),
        ],
        out_specs=[
            pl.BlockSpec((RB, H), lambda r: (r, 0)),
            pl.BlockSpec((G, H), lambda r: (0, 0)),
        ],
        out_shape=[
            jax.ShapeDtypeStruct((N, H), jnp.float32),
            jax.ShapeDtypeStruct((G, H), jnp.float32),
        ],
    )(z2, s2, g2, be2, batch3)


def _final_body(p5_ref, lw5_ref, lb5_ref, b3_ref, o_ref):
    acc = jnp.zeros((G, T), jnp.float32)
    for i in range(5):
        acc = acc + _mm(p5_ref[i], lw5_ref[i])
    cnt = jnp.zeros((G, 1), jnp.float32)
    for r in range(NRB):
        bb = b3_ref[r]
        mT = (lax.broadcasted_iota(jnp.int32, (G, RB), 0) == bb
              ).astype(jnp.float32)
        cnt = cnt + jnp.sum(mT, axis=1, keepdims=True)
    acc = acc + cnt * lb5_ref[0]
    acc = acc + (lb5_ref[1] + lb5_ref[2] + lb5_ref[3] + lb5_ref[4])
    o_ref[...] = acc


def _final(p5, lw5, lb5, batch3):
    return pl.pallas_call(
        _final_body,
        in_specs=[
            pl.BlockSpec((5, G, H), lambda: (0, 0, 0)),
            pl.BlockSpec((5, H, T), lambda: (0, 0, 0)),
            pl.BlockSpec((5, 1, T), lambda: (0, 0, 0)),
            pl.BlockSpec((NRB, 1, RB), lambda: (0, 0, 0)),
        ],
        out_specs=pl.BlockSpec((G, T), lambda: (0, 0)),
        out_shape=jax.ShapeDtypeStruct((G, T), jnp.float32),
    )(p5, lw5, lb5, batch3)


def _sc_agg(h2d, sidx4, didx):
    mesh = plsc.VectorSubcoreMesh(core_axis_name="c", subcore_axis_name="s")

    @functools.partial(
        pl.kernel, mesh=mesh,
        out_type=jax.ShapeDtypeStruct((NCH, ACC_ROWS, CH), jnp.float32),
        scratch_types=[
            pltpu.VMEM((SB, EB), jnp.int32),
            pltpu.VMEM((SB, EB), jnp.int32),
            pltpu.VMEM((EB, CH), jnp.float32),
            pltpu.VMEM((EB, CH), jnp.float32),
            pltpu.SemaphoreType.DMA,
            pltpu.SemaphoreType.DMA,
            pltpu.SemaphoreType.DMA,
            pltpu.SemaphoreType.DMA,
            pltpu.VMEM_SHARED((ACC_ROWS, CH), jnp.float32),
        ],
    )
    def k(h_hbm, sidx_hbm, didx_hbm, out_hbm,
          sbuf, dbuf, rbuf0, rbuf1, sem0, sem1, gsem0, gsem1, acc):
        cid = lax.axis_index("c")
        sid = lax.axis_index("s")
        rbufs = (rbuf0, rbuf1)
        sems = (sem0, sem1)
        gsems = (gsem0, gsem1)

        def wait_scatter(b):
            pltpu.make_async_copy(rbufs[b], acc.at[dbuf.at[0]],
                                  sems[b]).wait()

        def wait_gather(b):
            pltpu.make_async_copy(h_hbm.at[sbuf.at[0]], rbufs[b],
                                  gsems[b]).wait()

        for cc in range(2):
            chunk = cid * 2 + cc
            base = sid * ZROWS

            @pl.loop(0, EB)
            def _(i):
                for kk in range(CH // 16):
                    rbuf0[i, pl.ds(kk * 16, 16)] = jnp.zeros((16,),
                                                             jnp.float32)
            for kk in range(ZROWS // EB):
                pltpu.sync_copy(rbuf0, acc.at[pl.ds(base + kk * EB, EB)])
            plsc.subcore_barrier()

            for st in range(NST):
                pltpu.sync_copy(sidx_hbm.at[sid].at[pl.ds(st * SB, SB)],
                                sbuf)
                pltpu.sync_copy(didx_hbm.at[sid].at[pl.ds(st * SB, SB)],
                                dbuf)

                @pl.loop(0, SB)
                def _(j):
                    for kk in range(EB // 16):
                        sbuf[j, pl.ds(kk * 16, 16)] = (
                            sbuf[j, pl.ds(kk * 16, 16)] + chunk)

                pltpu.async_copy(h_hbm.at[sbuf.at[0]], rbuf0, gsem0)

                def step(j, u, prefetch, guard_first=False):
                    b = u % 2
                    nb = 1 - b
                    wait_gather(b)
                    if prefetch:
                        if guard_first:
                            @pl.when(j > 0)
                            def _():
                                wait_scatter(nb)
                        else:
                            wait_scatter(nb)
                        pltpu.async_copy(h_hbm.at[sbuf.at[j + 1]],
                                         rbufs[nb], gsems[nb])
                    pltpu.async_copy(rbufs[b], acc.at[dbuf.at[j]],
                                     sems[b], add=True)

                @pl.loop(0, SB // 4 - 1)
                def _(i):
                    for u in range(4):
                        step(i * 4 + u, u, True, guard_first=(u == 0))

                for u in range(4):
                    step(SB - 4 + u, u, u < 3)

                wait_scatter(0)
                wait_scatter(1)

            plsc.subcore_barrier()
            pltpu.sync_copy(
                acc.at[pl.ds(sid * ZROWS, ZROWS)],
                out_hbm.at[chunk].at[pl.ds(sid * ZROWS, ZROWS)])
            plsc.subcore_barrier()

    return k(h2d, sidx4, didx)


def _mlp0(x, p, batch3):
    z1, s1 = _passA0(x, p["w1"], p["b1"].reshape(1, H))
    z2, s2 = _passB(z1, s1, p["g1"].reshape(1, H), p["be1"].reshape(1, H),
                    p["w2"], p["b2"].reshape(1, H))
    return _passC(z2, s2, p["g2"].reshape(1, H), p["be2"].reshape(1, H),
                  batch3)


def _mlp(h, agg4, p, batch3):
    z1, s1 = _passA(h, agg4, p["w1"], p["b1"].reshape(1, H))
    z2, s2 = _passB(z1, s1, p["g1"].reshape(1, H), p["be1"].reshape(1, H),
                    p["w2"], p["b2"].reshape(1, H))
    return _passC(z2, s2, p["g2"].reshape(1, H), p["be2"].reshape(1, H),
                  batch3)


def kernel(x, edge_index, batch, params):
    src = edge_index[0].astype(jnp.int32)
    dst = edge_index[1].astype(jnp.int32)
    npad = EPAD - EPW
    ar = jnp.arange(npad, dtype=jnp.int32)[None, :]
    aw = jnp.arange(NSUB, dtype=jnp.int32)[:, None]
    pad_src = (ar * 163 + aw * 613) % N
    pad_dst = N + (ar + 15 * aw) % (ACC_ROWS - N)
    srcp = jnp.concatenate([src.reshape(NSUB, EPW), pad_src], axis=1)
    dstp = jnp.concatenate([dst.reshape(NSUB, EPW), pad_dst], axis=1)
    sidx4 = (srcp * NCH).reshape(NSUB, NEB, EB)
    didx = dstp.reshape(NSUB, NEB, EB)
    batch3 = batch.astype(jnp.int32).reshape(NRB, 1, RB)

    h, pooled0 = _mlp0(x, params["first_h"], batch3)
    pooled = [pooled0]
    for i in range(4):
        agg4 = _sc_agg(h.reshape(N * NCH, CH), sidx4, didx)
        h, p_i = _mlp(h, agg4, params["nns"][i], batch3)
        pooled.append(p_i)

    p5 = jnp.stack(pooled)
    lw5 = jnp.stack(params["lin_w"])
    lb5 = jnp.stack(params["lin_b"]).reshape(5, 1, T)
    return _final(p5, lw5, lb5, batch3)
